# baseline (speedup 1.0000x reference)
import functools
import math

import jax
import jax.numpy as jnp
from jax import lax
from jax.experimental import pallas as pl
from jax.experimental.pallas import tpu as pltpu
from jax.experimental.pallas import tpu_sc as plsc

DIM = 128
SCALE = math.sqrt(float(DIM))

_NC = 2
_NS = 16
_NW = _NC * _NS


@functools.lru_cache(maxsize=None)
def _make_kernel(batch, hist):
    b_per_w = batch // _NW
    chunk_b = 2
    chunk = chunk_b * hist
    assert batch % _NW == 0 and chunk <= 128
    n_chunks = b_per_w // chunk_b
    assert n_chunks % 2 == 0 and n_chunks >= 4
    mesh = plsc.VectorSubcoreMesh(core_axis_name="c", subcore_axis_name="s")

    @functools.partial(
        pl.kernel,
        mesh=mesh,
        out_type=jax.ShapeDtypeStruct((batch, hist, DIM), jnp.float32),
        scratch_types=[
            pltpu.VMEM((n_chunks, chunk), jnp.int32),
            pltpu.VMEM((2, chunk, DIM), jnp.float32),
            pltpu.SemaphoreType.DMA,
            pltpu.SemaphoreType.DMA,
            pltpu.SemaphoreType.DMA,
            pltpu.SemaphoreType.DMA,
        ],
    )
    def body(idx_hbm, table_hbm, out_hbm, idx_v, rows_v, g0, g1, o0, o1):
        wid = lax.axis_index("s") * _NC + lax.axis_index("c")
        b0 = wid * b_per_w
        pltpu.sync_copy(idx_hbm.at[wid], idx_v)

        gsem = (g0, g1)
        osem = (o0, o1)

        def g_copy(j, b):
            return pltpu.make_async_copy(
                table_hbm.at[idx_v.at[j]], rows_v.at[b], gsem[b])

        def o_copies(j, b):
            bb = b0 + chunk_b * j
            return [
                pltpu.make_async_copy(
                    rows_v.at[b, pl.ds(u * hist, hist)],
                    out_hbm.at[bb + u], osem[b])
                for u in range(chunk_b)
            ]

        def scale(b):
            @plsc.parallel_loop(0, chunk, step=1, unroll=4)
            def _sb(i):
                for u in range(DIM // 16):
                    sl = pl.ds(u * 16, 16)
                    rows_v[b, i, sl] = rows_v[b, i, sl] * SCALE

        def step(j, b, first, last):
            g_copy(j, b).wait()
            if not first:
                for c in o_copies(j - 1, 1 - b):
                    c.wait()
            if not last:
                g_copy(j + 1, 1 - b).start()
            scale(b)
            for c in o_copies(j, b):
                c.start()

        g_copy(0, 0).start()
        step(0, 0, True, False)
        step(1, 1, False, False)

        def loop_body(jp, c):
            step(2 * jp, 0, False, False)
            step(2 * jp + 1, 1, False, False)
            return c

        lax.fori_loop(1, n_chunks // 2 - 1, loop_body, 0)

        step(n_chunks - 2, 0, False, False)
        step(n_chunks - 1, 1, False, True)
        for c in o_copies(n_chunks - 1, 1):
            c.wait()

    return body


def kernel(input, table):
    batch, hist = input.shape
    half = batch // 2
    b_per_w = half // _NW
    k = _make_kernel(half, hist)

    def run(part):
        idx3 = part.reshape(_NW, b_per_w // 2, 2 * hist)
        return k(idx3, table)

    o1 = run(input[:half])
    o2 = run(input[half:])
    return jnp.concatenate([o1, o2], axis=0)


# speedup vs baseline: 1.2336x; 1.2336x over previous
import functools
import math

import jax
import jax.numpy as jnp
from jax import lax
from jax.experimental import pallas as pl
from jax.experimental.pallas import tpu as pltpu
from jax.experimental.pallas import tpu_sc as plsc

DIM = 128
SCALE = math.sqrt(float(DIM))

_NC = 2
_NS = 16
_NW = _NC * _NS


@functools.lru_cache(maxsize=None)
def _make_kernel(batch, hist):
    b_per_w = batch // _NW
    assert batch % _NW == 0 and hist <= 128
    assert b_per_w % 2 == 0 and b_per_w >= 4
    mesh = plsc.VectorSubcoreMesh(core_axis_name="c", subcore_axis_name="s")

    @functools.partial(
        pl.kernel,
        mesh=mesh,
        out_type=jax.ShapeDtypeStruct((batch, hist, DIM), jnp.float32),
        scratch_types=[
            pltpu.VMEM((b_per_w, hist), jnp.int32),
            pltpu.VMEM((2, hist, DIM), jnp.float32),
            pltpu.SemaphoreType.DMA,
            pltpu.SemaphoreType.DMA,
            pltpu.SemaphoreType.DMA,
            pltpu.SemaphoreType.DMA,
        ],
        compiler_params=pltpu.CompilerParams(use_tc_tiling_on_sc=True),
    )
    def body(idx_hbm, table_hbm, out_hbm, idx_v, rows_v, g0, g1, o0, o1):
        wid = lax.axis_index("s") * _NC + lax.axis_index("c")
        b0 = wid * b_per_w
        pltpu.sync_copy(idx_hbm.at[pl.ds(b0, b_per_w)], idx_v)

        gsem = (g0, g1)
        osem = (o0, o1)

        def g_copy(j, b):
            return pltpu.make_async_copy(
                table_hbm.at[idx_v.at[j]], rows_v.at[b], gsem[b])

        def o_copy(j, b):
            return pltpu.make_async_copy(
                rows_v.at[b], out_hbm.at[b0 + j], osem[b])

        def scale(b):
            @plsc.parallel_loop(0, hist, step=1, unroll=2)
            def _sb(i):
                for u in range(DIM // 16):
                    sl = pl.ds(u * 16, 16)
                    rows_v[b, i, sl] = rows_v[b, i, sl] * SCALE

        def step(j, b, first, last):
            g_copy(j, b).wait()
            if not first:
                o_copy(j - 1, 1 - b).wait()
            if not last:
                g_copy(j + 1, 1 - b).start()
            scale(b)
            o_copy(j, b).start()

        g_copy(0, 0).start()
        step(0, 0, True, False)
        step(1, 1, False, False)

        def loop_body(jp, c):
            step(2 * jp, 0, False, False)
            step(2 * jp + 1, 1, False, False)
            return c

        lax.fori_loop(1, b_per_w // 2 - 1, loop_body, 0)

        step(b_per_w - 2, 0, False, False)
        step(b_per_w - 1, 1, False, True)
        o_copy(b_per_w - 1, 1).wait()

    return body


def kernel(input, table):
    batch, hist = input.shape
    return _make_kernel(batch, hist)(input, table)


# speedup vs baseline: 2.6707x; 2.1649x over previous
import functools
import math

import jax
import jax.numpy as jnp
from jax import lax
from jax.experimental import pallas as pl
from jax.experimental.pallas import tpu as pltpu
from jax.experimental.pallas import tpu_sc as plsc

DIM = 128
SCALE = math.sqrt(float(DIM))

_NC = 2
_NS = 16
_NW = _NC * _NS


@functools.lru_cache(maxsize=None)
def _make_kernel(batch, hist):
    b_per_w = batch // _NW
    assert batch % _NW == 0 and b_per_w % 8 == 0 and b_per_w <= 128
    n_chunks = hist
    assert n_chunks % 2 == 0 and n_chunks >= 6
    mesh = plsc.VectorSubcoreMesh(core_axis_name="c", subcore_axis_name="s")

    @functools.partial(
        pl.kernel,
        mesh=mesh,
        out_type=jax.ShapeDtypeStruct((hist, batch, DIM), jnp.float32),
        scratch_types=[
            pltpu.VMEM((n_chunks, b_per_w), jnp.int32),
            pltpu.VMEM((2, b_per_w, DIM), jnp.float32),
            pltpu.SemaphoreType.DMA,
            pltpu.SemaphoreType.DMA,
            pltpu.SemaphoreType.DMA,
            pltpu.SemaphoreType.DMA,
        ],
    )
    def body(idx_hbm, table_hbm, out_hbm, idx_v, rows_v, g0, g1, o0, o1):
        wid = lax.axis_index("s") * _NC + lax.axis_index("c")
        b0 = wid * b_per_w
        pltpu.sync_copy(idx_hbm.at[wid], idx_v)

        gsem = (g0, g1)
        osem = (o0, o1)

        def g_copy(j, b):
            return pltpu.make_async_copy(
                table_hbm.at[idx_v.at[j]], rows_v.at[b], gsem[b])

        def o_copy(j, b):
            return pltpu.make_async_copy(
                rows_v.at[b], out_hbm.at[j, pl.ds(b0, b_per_w)], osem[b])

        def scale(b):
            @plsc.parallel_loop(0, b_per_w, step=1, unroll=4)
            def _sb(i):
                for u in range(DIM // 16):
                    sl = pl.ds(u * 16, 16)
                    rows_v[b, i, sl] = rows_v[b, i, sl] * SCALE

        def step(j, b, first, last):
            g_copy(j, b).wait()
            if not first:
                o_copy(j - 1, 1 - b).wait()
            if not last:
                g_copy(j + 1, 1 - b).start()
            scale(b)
            o_copy(j, b).start()

        g_copy(0, 0).start()
        step(0, 0, True, False)
        step(1, 1, False, False)

        def loop_body(jp, c):
            step(2 * jp, 0, False, False)
            step(2 * jp + 1, 1, False, False)
            return c

        lax.fori_loop(1, n_chunks // 2 - 1, loop_body, 0)

        step(n_chunks - 2, 0, False, False)
        step(n_chunks - 1, 1, False, True)
        o_copy(n_chunks - 1, 1).wait()

    return body


def kernel(input, table):
    batch, hist = input.shape
    b_per_w = batch // _NW
    idx3 = input.T.reshape(hist, _NW, b_per_w).transpose(1, 0, 2)
    out_t = _make_kernel(batch, hist)(idx3, table)
    return out_t.transpose(1, 0, 2)


# speedup vs baseline: 3.0860x; 1.1555x over previous
import functools
import math

import jax
import jax.numpy as jnp
from jax import lax
from jax.experimental import pallas as pl
from jax.experimental.pallas import tpu as pltpu
from jax.experimental.pallas import tpu_sc as plsc

DIM = 128
SCALE = math.sqrt(float(DIM))

_NC = 2
_NS = 16
_NW = _NC * _NS


@functools.lru_cache(maxsize=None)
def _make_kernel(batch, hist):
    b_per_w = batch // _NW
    assert batch % _NW == 0 and b_per_w % 8 == 0 and b_per_w <= 128
    n_chunks = hist
    assert n_chunks % 2 == 0 and n_chunks >= 6
    mesh = plsc.VectorSubcoreMesh(core_axis_name="c", subcore_axis_name="s")

    @functools.partial(
        pl.kernel,
        mesh=mesh,
        out_type=jax.ShapeDtypeStruct((hist, batch, DIM), jnp.float32),
        scratch_types=[
            pltpu.VMEM((n_chunks, b_per_w), jnp.int32),
            pltpu.VMEM((4, b_per_w, DIM), jnp.float32),
            [pltpu.SemaphoreType.DMA] * 4,
            [pltpu.SemaphoreType.DMA] * 4,
        ],
    )
    def body(idx_hbm, table_hbm, out_hbm, idx_v, rows_v, gsem, osem):
        wid = lax.axis_index("s") * _NC + lax.axis_index("c")
        b0 = wid * b_per_w
        pltpu.sync_copy(idx_hbm.at[wid], idx_v)

        def g_copy(j, b):
            return pltpu.make_async_copy(
                table_hbm.at[idx_v.at[j]], rows_v.at[b], gsem[b])

        def o_copy(j, b):
            return pltpu.make_async_copy(
                rows_v.at[b], out_hbm.at[j, pl.ds(b0, b_per_w)], osem[b])

        def scale(b):
            @plsc.parallel_loop(0, b_per_w, step=1, unroll=4)
            def _sb(i):
                for u in range(DIM // 16):
                    sl = pl.ds(u * 16, 16)
                    rows_v[b, i, sl] = rows_v[b, i, sl] * SCALE

        def step(j, b, refill, drain):
            g_copy(j, b).wait()
            scale(b)
            o_copy(j, b).start()
            if refill:
                if drain:
                    o_copy(j - 1, (b + 3) % 4).wait()
                g_copy(j + 3, (b + 3) % 4).start()

        for k in range(3):
            g_copy(k, k).start()
        step(0, 0, True, False)
        step(1, 1, True, True)

        def loop_body(jp, c):
            j = 2 + 4 * jp
            for h in range(4):
                step(j + h, (2 + h) % 4, True, True)
            return c

        lax.fori_loop(0, (n_chunks - 6) // 4, loop_body, 0)

        step(n_chunks - 4, (n_chunks - 4) % 4, True, True)
        for j in range(n_chunks - 3, n_chunks):
            step(j, j % 4, False, False)
        for j in range(n_chunks - 4, n_chunks):
            o_copy(j, j % 4).wait()

    return body


def kernel(input, table):
    batch, hist = input.shape
    b_per_w = batch // _NW
    idx3 = input.T.reshape(hist, _NW, b_per_w).transpose(1, 0, 2)
    out_t = _make_kernel(batch, hist)(idx3, table)
    return out_t.transpose(1, 0, 2)


# speedup vs baseline: 3.0992x; 1.0043x over previous
import functools
import math

import jax
import jax.numpy as jnp
from jax import lax
from jax.experimental import pallas as pl
from jax.experimental.pallas import tpu as pltpu
from jax.experimental.pallas import tpu_sc as plsc

DIM = 128
SCALE = math.sqrt(float(DIM))

_NC = 2
_NS = 16
_NW = _NC * _NS
_NBUF = 6


@functools.lru_cache(maxsize=None)
def _make_kernel(batch, hist):
    b_per_w = batch // _NW
    assert batch % _NW == 0 and b_per_w % 8 == 0 and b_per_w <= 128
    n_chunks = hist
    assert n_chunks % 2 == 0 and n_chunks >= 6
    mesh = plsc.VectorSubcoreMesh(core_axis_name="c", subcore_axis_name="s")

    @functools.partial(
        pl.kernel,
        mesh=mesh,
        out_type=jax.ShapeDtypeStruct((hist, batch, DIM), jnp.float32),
        scratch_types=[
            pltpu.VMEM((n_chunks, b_per_w), jnp.int32),
            pltpu.VMEM((_NBUF, b_per_w, DIM), jnp.float32),
            [pltpu.SemaphoreType.DMA] * _NBUF,
            [pltpu.SemaphoreType.DMA] * _NBUF,
        ],
    )
    def body(idx_hbm, table_hbm, out_hbm, idx_v, rows_v, gsem, osem):
        wid = lax.axis_index("s") * _NC + lax.axis_index("c")
        b0 = wid * b_per_w
        pltpu.sync_copy(idx_hbm.at[wid], idx_v)

        def g_copy(j, b):
            return pltpu.make_async_copy(
                table_hbm.at[idx_v.at[j]], rows_v.at[b], gsem[b])

        def o_copy(j, b):
            return pltpu.make_async_copy(
                rows_v.at[b], out_hbm.at[j, pl.ds(b0, b_per_w)], osem[b])

        def scale(b):
            @plsc.parallel_loop(0, b_per_w, step=1, unroll=4)
            def _sb(i):
                for u in range(DIM // 16):
                    sl = pl.ds(u * 16, 16)
                    rows_v[b, i, sl] = rows_v[b, i, sl] * SCALE

        def step(j, b, refill, drain):
            g_copy(j, b).wait()
            scale(b)
            o_copy(j, b).start()
            if refill:
                if drain:
                    o_copy(j - 1, (b - 1) % _NBUF).wait()
                g_copy(j + _NBUF - 1, (b - 1) % _NBUF).start()

        for k in range(_NBUF - 1):
            g_copy(k, k).start()
        step(0, 0, True, False)

        n_steady = n_chunks - _NBUF
        n_main = (n_steady // _NBUF) * _NBUF

        def loop_body(jp, c):
            j = 1 + _NBUF * jp
            for h in range(_NBUF):
                step(j + h, (1 + h) % _NBUF, True, True)
            return c

        lax.fori_loop(0, n_main // _NBUF, loop_body, 0)

        for j in range(1 + n_main, n_steady + 1):
            step(j, j % _NBUF, True, True)
        for j in range(n_steady + 1, n_chunks):
            step(j, j % _NBUF, False, False)
        for j in range(n_chunks - _NBUF, n_chunks):
            o_copy(j, j % _NBUF).wait()

    return body


def kernel(input, table):
    batch, hist = input.shape
    b_per_w = batch // _NW
    idx3 = input.T.reshape(hist, _NW, b_per_w).transpose(1, 0, 2)
    out_t = _make_kernel(batch, hist)(idx3, table)
    return out_t.transpose(1, 0, 2)
